# Initial kernel scaffold; baseline (speedup 1.0000x reference)
import jax
import jax.numpy as jnp
from jax.experimental import pallas as pl


def kernel(index, length, data):
    raise NotImplementedError("write your pallas kernel here")


# speedup vs baseline: 4.1619x; 4.1619x over previous
import functools

import jax
import jax.numpy as jnp
from jax import lax
from jax.experimental import pallas as pl
from jax.experimental.pallas import tpu as pltpu
from jax.experimental.pallas import tpu_sc as plsc

P = 100000
B = 4096
T = 200
C = 32

NC = 2
NS = 16
NW = NC * NS
BPW = B // NW
ROWS_PW = BPW * T
CHUNK = 128
NCHUNK = ROWS_PW // CHUNK
GIDX_LEN = ROWS_PW + 64

_mesh = plsc.VectorSubcoreMesh(core_axis_name="c", subcore_axis_name="s")


@functools.partial(
    pl.kernel,
    mesh=_mesh,
    out_type=jax.ShapeDtypeStruct((B * T, C), jnp.float32),
    scratch_types=[
        pltpu.VMEM((BPW,), jnp.int32),
        pltpu.VMEM((16,), jnp.int32),
        pltpu.VMEM((GIDX_LEN,), jnp.int32),
        pltpu.VMEM((CHUNK, C), jnp.float32),
        pltpu.VMEM((CHUNK, C), jnp.float32),
        pltpu.SemaphoreType.DMA,
        pltpu.SemaphoreType.DMA,
    ],
    compiler_params=pltpu.CompilerParams(
        needs_layout_passes=False, use_tc_tiling_on_sc=False
    ),
)
def _sc_gather(idx_hbm, shift_hbm, data_hbm, out_hbm,
               idx_v, shift_v, gidx_v, buf0, buf1, gs0, gs1):
    wid = lax.axis_index("s") * NC + lax.axis_index("c")
    base_b = wid * BPW

    pltpu.sync_copy(idx_hbm.at[pl.ds(base_b, BPW)], idx_v)
    pltpu.sync_copy(shift_hbm, shift_v)

    shift_vec = shift_v[...]
    iota = lax.iota(jnp.int32, 16)

    def build_b(b, carry):
        bb = jnp.full((16,), 0, jnp.int32) + b
        base = plsc.load_gather(idx_v, [bb]) + shift_vec
        f0 = b * T
        for j in range(13):
            v = base + (j * 16) + iota
            v = jnp.where(v >= P, v - P, v)
            v = jnp.where(v < 0, v + P, v)
            gidx_v[pl.ds(f0 + j * 16, 16)] = v
        return carry

    lax.fori_loop(0, BPW, build_b, 0)

    wbase = wid * ROWS_PW

    def pipe(c, carry):
        pltpu.async_copy(
            data_hbm.at[gidx_v.at[pl.ds(c * CHUNK, CHUNK)]], buf0, gs0
        ).wait()
        pltpu.sync_copy(buf0, out_hbm.at[pl.ds(wbase + c * CHUNK, CHUNK)])
        return carry

    lax.fori_loop(0, NCHUNK, pipe, 0)


def kernel(index, length, data):
    shift = jnp.broadcast_to(
        (jnp.asarray(length, jnp.int32) - T).reshape(()), (16,)
    ).astype(jnp.int32)
    out = _sc_gather(index.astype(jnp.int32), shift, data)
    return out.reshape(B, T, C)


# speedup vs baseline: 5.1837x; 1.2455x over previous
import functools

import jax
import jax.numpy as jnp
from jax import lax
from jax.experimental import pallas as pl
from jax.experimental.pallas import tpu as pltpu
from jax.experimental.pallas import tpu_sc as plsc

P = 100000
B = 4096
T = 200
C = 32

NC = 2
NS = 16
NW = NC * NS
BPW = B // NW
ROWS_PW = BPW * T
CHUNK = 128
NCHUNK = ROWS_PW // CHUNK
GIDX_LEN = ROWS_PW + 64

_mesh = plsc.VectorSubcoreMesh(core_axis_name="c", subcore_axis_name="s")


@functools.partial(
    pl.kernel,
    mesh=_mesh,
    out_type=jax.ShapeDtypeStruct((B * T, C), jnp.float32),
    scratch_types=[
        pltpu.VMEM((BPW,), jnp.int32),
        pltpu.VMEM((16,), jnp.int32),
        pltpu.VMEM((GIDX_LEN,), jnp.int32),
        [pltpu.VMEM((CHUNK, C), jnp.float32) for _ in range(4)],
        [pltpu.SemaphoreType.DMA for _ in range(4)],
        [pltpu.SemaphoreType.DMA for _ in range(4)],
    ],
    compiler_params=pltpu.CompilerParams(
        needs_layout_passes=False, use_tc_tiling_on_sc=False
    ),
)
def _sc_gather(idx_hbm, shift_hbm, data_hbm, out_hbm,
               idx_v, shift_v, gidx_v, bufs, gsems, wsems):
    wid = lax.axis_index("s") * NC + lax.axis_index("c")
    base_b = wid * BPW

    pltpu.sync_copy(idx_hbm.at[pl.ds(base_b, BPW)], idx_v)
    pltpu.sync_copy(shift_hbm, shift_v)

    shift_vec = shift_v[...]
    iota = lax.iota(jnp.int32, 16)

    def build_b(b, carry):
        bb = jnp.full((16,), 0, jnp.int32) + b
        base = plsc.load_gather(idx_v, [bb]) + shift_vec
        f0 = b * T
        for j in range(13):
            v = base + (j * 16) + iota
            v = jnp.where(v >= P, v - P, v)
            v = jnp.where(v < 0, v + P, v)
            gidx_v[pl.ds(f0 + j * 16, 16)] = v
        return carry

    lax.fori_loop(0, BPW, build_b, 0)

    wbase = wid * ROWS_PW

    def start_gather(c, s):
        pltpu.async_copy(
            data_hbm.at[gidx_v.at[pl.ds(c * CHUNK, CHUNK)]], bufs[s], gsems[s]
        )

    def wait_gather(c, s):
        pltpu.make_async_copy(
            data_hbm.at[gidx_v.at[pl.ds(c * CHUNK, CHUNK)]], bufs[s], gsems[s]
        ).wait()

    def start_write(c, s):
        pltpu.async_copy(
            bufs[s], out_hbm.at[pl.ds(wbase + c * CHUNK, CHUNK)], wsems[s]
        )

    def wait_write(c, s):
        pltpu.make_async_copy(
            bufs[s], out_hbm.at[pl.ds(wbase + c * CHUNK, CHUNK)], wsems[s]
        ).wait()

    start_gather(0, 0)
    start_gather(1, 1)
    wait_gather(0, 0)
    start_write(0, 0)
    start_gather(2, 2)
    wait_gather(1, 1)
    start_write(1, 1)
    start_gather(3, 3)

    def pipe(g, carry):
        for k in range(4):
            c = 2 + 4 * g + k
            wait_write(c - 2, k)
            start_gather(c + 2, k)
            wait_gather(c, (k + 2) % 4)
            start_write(c, (k + 2) % 4)
        return carry

    lax.fori_loop(0, (NCHUNK - 4) // 4, pipe, 0)

    wait_gather(NCHUNK - 2, 2)
    start_write(NCHUNK - 2, 2)
    wait_gather(NCHUNK - 1, 3)
    start_write(NCHUNK - 1, 3)
    for k in range(4):
        wait_write(NCHUNK - 4 + k, k)


def kernel(index, length, data):
    shift = jnp.broadcast_to(
        (jnp.asarray(length, jnp.int32) - T).reshape(()), (16,)
    ).astype(jnp.int32)
    out = _sc_gather(index.astype(jnp.int32), shift, data)
    return out.reshape(B, T, C)
